# baseline (speedup 1.0000x reference)
import dataclasses

import jax
import jax.numpy as jnp
from jax.experimental import pallas as pl
from jax.experimental.pallas import tpu as pltpu
from jax.experimental.pallas import tpu_sc as plsc

B, S, D = 16, 4096, 128
SBLK = 1024


def _mm_kernel(x_ref, m_ref, w_ref, o_ref, cand_ref, lens_ref,
               lacc_ref, cacc_ref):
    k = pl.program_id(0)
    nsteps = pl.num_programs(0)

    @pl.when(k == 0)
    def _init():
        lacc_ref[...] = jnp.zeros_like(lacc_ref)
        cacc_ref[...] = jnp.zeros_like(cacc_ref)

    x = x_ref[...]
    m = m_ref[...]
    w = w_ref[...]
    y = jnp.dot(x.reshape(B * SBLK, D), w,
                preferred_element_type=jnp.float32).reshape(B, SBLK, D)
    o_ref[...] = y * m[:, :, None]

    local_len = jnp.sum(m, axis=1)
    lacc_ref[...] = lacc_ref[...] + local_len[None, :]
    m_next = jnp.concatenate([m[:, 1:], jnp.zeros((B, 1), m.dtype)], axis=1)
    flag = m * (1.0 - m_next)
    contrib = jnp.sum(flag[:, :, None] * y, axis=1)
    has = jnp.sum(flag, axis=1)[:, None] > 0.0
    cacc_ref[...] = jnp.where(has, contrib, cacc_ref[...])

    @pl.when(k == nsteps - 1)
    def _emit():
        lens_ref[...] = lacc_ref[...]
        cand_ref[...] = cacc_ref[...]


def _sc_compiler_params():
    cp = pltpu.CompilerParams()
    if "needs_layout_passes" in pltpu.CompilerParams.__dataclass_fields__:
        cp = dataclasses.replace(cp, needs_layout_passes=False)
    return cp


def _sc_fin_kernel(lens_hbm, cand_hbm, rinv_hbm, fin_hbm,
                   lbuf, cbuf, ribuf, gbuf, sem):
    c = jax.lax.axis_index("c")
    s = jax.lax.axis_index("s")

    @pl.when((c == 0) & (s == 0))
    def _finish():
        l_copy = pltpu.make_async_copy(lens_hbm.at[0], lbuf, sem)
        c_copy = pltpu.make_async_copy(cand_hbm, cbuf, sem)
        l_copy.start()
        c_copy.start()
        l_copy.wait()
        c_copy.wait()
        lens = lbuf[...].astype(jnp.int32)
        iota = jax.lax.iota(jnp.int32, 16)
        keys = lens * 16 + (15 - iota)
        _, perm = plsc.sort_key_val(keys, iota, descending=True)
        _, ri = plsc.sort_key_val(perm, iota)
        nv = plsc.all_reduce_population_count(lens >= 1)
        ribuf[...] = ri * 65536 + nv
        for col in range(D):
            cidx = jnp.full((16,), col, jnp.int32)
            v = plsc.load_gather(cbuf, [perm, cidx])
            plsc.store_scatter(gbuf, [iota, cidx], v)
        r_copy = pltpu.make_async_copy(ribuf, rinv_hbm, sem)
        g_copy = pltpu.make_async_copy(gbuf, fin_hbm, sem)
        r_copy.start()
        g_copy.start()
        r_copy.wait()
        g_copy.wait()


@jax.jit
def kernel(inputs, mask, W):
    restored, cand, lens = pl.pallas_call(
        _mm_kernel,
        grid=(S // SBLK,),
        in_specs=[
            pl.BlockSpec((B, SBLK, D), lambda k: (0, k, 0)),
            pl.BlockSpec((B, SBLK), lambda k: (0, k)),
            pl.BlockSpec((D, D), lambda k: (0, 0)),
        ],
        out_specs=[
            pl.BlockSpec((B, SBLK, D), lambda k: (0, k, 0)),
            pl.BlockSpec((B, D), lambda k: (0, 0)),
            pl.BlockSpec((1, B), lambda k: (0, 0)),
        ],
        out_shape=[
            jax.ShapeDtypeStruct((B, S, D), jnp.float32),
            jax.ShapeDtypeStruct((B, D), jnp.float32),
            jax.ShapeDtypeStruct((1, B), jnp.float32),
        ],
        scratch_shapes=[
            pltpu.VMEM((1, B), jnp.float32),
            pltpu.VMEM((B, D), jnp.float32),
        ],
    )(inputs, mask, W)

    sc_fin = pl.kernel(
        _sc_fin_kernel,
        out_type=[
            jax.ShapeDtypeStruct((16,), jnp.int32),
            jax.ShapeDtypeStruct((B, D), jnp.float32),
        ],
        mesh=plsc.VectorSubcoreMesh(core_axis_name="c", subcore_axis_name="s"),
        scratch_types=[
            pltpu.VMEM((16,), jnp.float32),
            pltpu.VMEM((B, D), jnp.float32),
            pltpu.VMEM((16,), jnp.int32),
            pltpu.VMEM((B, D), jnp.float32),
            pltpu.SemaphoreType.DMA,
        ],
        compiler_params=_sc_compiler_params(),
    )
    rinv, fin = sc_fin(lens, cand)

    ri = jax.lax.shift_right_logical(rinv, 16)
    nv = jax.lax.bitwise_and(rinv[0], 65535)
    return (restored, fin[None, :, :], ri, nv)


# speedup vs baseline: 1.1657x; 1.1657x over previous
import dataclasses

import jax
import jax.numpy as jnp
from jax.experimental import pallas as pl
from jax.experimental.pallas import tpu as pltpu
from jax.experimental.pallas import tpu_sc as plsc

B, S, D = 16, 4096, 128
SBLK = 1024


def _mm_kernel(x_ref, m_ref, w_ref, o_ref, lens_ref, lacc_ref):
    k = pl.program_id(0)
    nsteps = pl.num_programs(0)

    @pl.when(k == 0)
    def _init():
        lacc_ref[...] = jnp.zeros_like(lacc_ref)

    x = x_ref[...]
    m = m_ref[...]
    w = w_ref[...]
    y = jnp.dot(x.reshape(B * SBLK, D), w,
                preferred_element_type=jnp.float32).reshape(B, SBLK, D)
    o_ref[...] = y * m[:, :, None]
    lacc_ref[...] = lacc_ref[...] + jnp.sum(m, axis=1)[None, :]

    @pl.when(k == nsteps - 1)
    def _emit():
        lens_ref[...] = lacc_ref[...]


def _sc_compiler_params():
    cp = pltpu.CompilerParams()
    if "needs_layout_passes" in pltpu.CompilerParams.__dataclass_fields__:
        cp = dataclasses.replace(cp, needs_layout_passes=False)
    return cp


def _sc_fin_kernel(lens_hbm, r2d_hbm, rinv_hbm, fin_hbm,
                   lbuf, idxbuf, ribuf, gbuf, sem):
    s = jax.lax.axis_index("s")

    @pl.when(s == 0)
    def _finish():
        pltpu.async_copy(lens_hbm.at[0], lbuf, sem).wait()
        lens = lbuf[...].astype(jnp.int32)
        iota = jax.lax.iota(jnp.int32, 16)
        keys = lens * 16 + (15 - iota)
        keys_sorted, perm = plsc.sort_key_val(keys, iota, descending=True)
        _, ri = plsc.sort_key_val(perm, iota)
        lens_sorted = jax.lax.shift_right_logical(keys_sorted, 4)
        nv = plsc.all_reduce_population_count(lens >= 1)
        fidx = perm * S + jnp.maximum(lens_sorted - 1, 0)
        idxbuf[...] = fidx
        ribuf[...] = ri * 65536 + nv
        pltpu.sync_copy(r2d_hbm.at[idxbuf], gbuf)
        r_copy = pltpu.make_async_copy(ribuf, rinv_hbm, sem)
        g_copy = pltpu.make_async_copy(gbuf, fin_hbm, sem)
        r_copy.start()
        g_copy.start()
        r_copy.wait()
        g_copy.wait()


@jax.jit
def kernel(inputs, mask, W):
    restored, lens = pl.pallas_call(
        _mm_kernel,
        grid=(S // SBLK,),
        in_specs=[
            pl.BlockSpec((B, SBLK, D), lambda k: (0, k, 0)),
            pl.BlockSpec((B, SBLK), lambda k: (0, k)),
            pl.BlockSpec((D, D), lambda k: (0, 0)),
        ],
        out_specs=[
            pl.BlockSpec((B, SBLK, D), lambda k: (0, k, 0)),
            pl.BlockSpec((1, B), lambda k: (0, 0)),
        ],
        out_shape=[
            jax.ShapeDtypeStruct((B, S, D), jnp.float32),
            jax.ShapeDtypeStruct((1, B), jnp.float32),
        ],
        scratch_shapes=[
            pltpu.VMEM((1, B), jnp.float32),
        ],
    )(inputs, mask, W)

    sc_fin = pl.kernel(
        _sc_fin_kernel,
        out_type=[
            jax.ShapeDtypeStruct((16,), jnp.int32),
            jax.ShapeDtypeStruct((B, D), jnp.float32),
        ],
        mesh=plsc.VectorSubcoreMesh(core_axis_name="c", subcore_axis_name="s",
                                    num_cores=1),
        scratch_types=[
            pltpu.VMEM((16,), jnp.float32),
            pltpu.VMEM((16,), jnp.int32),
            pltpu.VMEM((16,), jnp.int32),
            pltpu.VMEM((B, D), jnp.float32),
            pltpu.SemaphoreType.DMA,
        ],
        compiler_params=_sc_compiler_params(),
    )
    rinv, fin = sc_fin(lens, restored.reshape(B * S, D))

    ri = jax.lax.shift_right_logical(rinv, 16)
    nv = jax.lax.bitwise_and(rinv[0], 65535)
    return (restored, fin[None, :, :], ri, nv)
